# baseline (speedup 1.0000x reference)
import functools

import jax
import jax.numpy as jnp
from jax import lax
from jax.experimental import pallas as pl
from jax.experimental.pallas import tpu as pltpu

B, N, DIN, H, L, K = 16, 100, 2, 128, 3, 10


def _gcn_body(node_ref, dem_ref, dis_ref, W1_ref, b1_ref, W23_ref, w3_ref,
              b23_ref, w4_ref, w5_ref, b45_ref, Wn_ref, We_ref,
              Wl_self_ref, bl_self_ref, Wl_nb_ref, bl_nb_ref,
              Wl_e_ref, bl_e_ref, Wl_from_ref, Wl_to_ref,
              x_out_ref, e_out_ref):
    f32 = jnp.float32
    dis = dis_ref[0]
    node = node_ref[0]
    dem = dem_ref[0]

    colid = lax.broadcasted_iota(jnp.int32, (N, N), 1)
    d = dis
    M = jnp.zeros((N, N), f32)
    for k in range(K + 1):
        m = jnp.min(d, axis=1, keepdims=True)
        cand = jnp.where(d == m, colid, N)
        j = jnp.min(cand, axis=1, keepdims=True)
        sel = colid == j
        d = jnp.where(sel, jnp.inf, d)
        if k > 0:
            M = M + sel.astype(f32)
    rowid = lax.broadcasted_iota(jnp.int32, (N, N), 0)
    eye = (rowid == colid).astype(f32)
    a = M * (1.0 - eye) - eye

    xd = jax.nn.relu(jnp.dot(node, W1_ref[...],
                             preferred_element_type=f32) + b1_ref[...])
    xc = jax.nn.relu(jnp.dot(node, W23_ref[...], preferred_element_type=f32)
                     + dem * w3_ref[...] + b23_ref[...])
    row0 = lax.broadcasted_iota(jnp.int32, (N, 1), 0) == 0
    x = jnp.where(row0, xd, xc)
    x = jnp.dot(x, Wn_ref[...], preferred_element_type=f32)

    bf16 = jnp.bfloat16
    e0 = jax.nn.relu(dis[:, :, None] * w4_ref[...][None]
                     + a[:, :, None] * w5_ref[...][None]
                     + b45_ref[...][None])
    e = jnp.dot(e0.reshape(N * N, H).astype(bf16),
                We_ref[...].astype(bf16),
                preferred_element_type=f32)

    inv_k = 1.0 / K
    for l in range(L):
        mx = jnp.dot(M, x, preferred_element_type=f32) * inv_k
        agg = jnp.dot(mx, Wl_nb_ref[l], preferred_element_type=f32) + bl_nb_ref[l]
        x = x + jax.nn.relu(
            jnp.dot(x, Wl_self_ref[l], preferred_element_type=f32)
            + bl_self_ref[l] + agg)
        xf = jnp.dot(x, Wl_from_ref[l], preferred_element_type=f32)
        xt = jnp.dot(x, Wl_to_ref[l], preferred_element_type=f32)
        et = jnp.dot(e.astype(bf16), Wl_e_ref[l].astype(bf16),
                     preferred_element_type=f32)
        e3 = et.reshape(N, N, H) + bl_e_ref[l][None, None] \
            + xf[:, None, :] + xt[None, :, :]
        e = e + jax.nn.relu(e3).reshape(N * N, H)

    x_out_ref[0] = x
    e_out_ref[0] = e.reshape(N, N, H)


def kernel(node, demand, dis, W1, b1, W2, b2, W3, b3, W4, b4, W5, b5, Wn, We,
           Wl_self, bl_self, Wl_nb, bl_nb, Wl_e, bl_e, Wl_from, Wl_to):
    f32 = jnp.float32
    Hh = H // 2
    W23 = jnp.zeros((DIN, H), f32).at[:, :Hh].set(W2)
    w3 = jnp.zeros((1, H), f32).at[0, Hh:].set(W3[0])
    b23 = jnp.concatenate([b2, b3])[None]
    w4 = jnp.concatenate([W4[0], jnp.zeros((Hh,), f32)])[None]
    w5 = jnp.concatenate([jnp.zeros((Hh,), f32), W5[0]])[None]
    b45 = jnp.concatenate([b4, b5])[None]

    dem3 = demand[..., None]
    rep = lambda shape: pl.BlockSpec(shape, lambda b: (0,) * len(shape))

    grid_spec = pl.GridSpec(
        grid=(B,),
        in_specs=[
            pl.BlockSpec((1, N, DIN), lambda b: (b, 0, 0)),
            pl.BlockSpec((1, N, 1), lambda b: (b, 0, 0)),
            pl.BlockSpec((1, N, N), lambda b: (b, 0, 0)),
            rep((DIN, H)), rep((1, H)), rep((DIN, H)), rep((1, H)),
            rep((1, H)), rep((1, H)), rep((1, H)), rep((1, H)),
            rep((H, H)), rep((H, H)),
            rep((L, H, H)), rep((L, H)), rep((L, H, H)), rep((L, H)),
            rep((L, H, H)), rep((L, H)), rep((L, H, H)), rep((L, H, H)),
        ],
        out_specs=[
            pl.BlockSpec((1, N, H), lambda b: (b, 0, 0)),
            pl.BlockSpec((1, N, N, H), lambda b: (b, 0, 0, 0)),
        ],
    )

    x_out, e_out = pl.pallas_call(
        _gcn_body,
        grid_spec=grid_spec,
        out_shape=[
            jax.ShapeDtypeStruct((B, N, H), f32),
            jax.ShapeDtypeStruct((B, N, N, H), f32),
        ],
        compiler_params=pltpu.CompilerParams(
            dimension_semantics=("arbitrary",),
        ),
    )(node, dem3, dis, W1, b1[None], W23, w3, b23, w4, w5, b45, Wn, We,
      Wl_self, bl_self, Wl_nb, bl_nb, Wl_e, bl_e, Wl_from, Wl_to)
    return (x_out, e_out)


# speedup vs baseline: 1.0718x; 1.0718x over previous
import jax
import jax.numpy as jnp
from jax import lax
from jax.experimental import pallas as pl
from jax.experimental.pallas import tpu as pltpu

B, N, DIN, H, L, K = 16, 100, 2, 128, 3, 10
NP = 104


def _gcn_body(node_ref, dem_ref, dis_ref, disf_ref, W1_ref, b1_ref, W23_ref,
              w3_ref, b23_ref, w4_ref, w5_ref, b45_ref, Wn_ref, We_ref,
              Wl_self_ref, bl_self_ref, Wl_nb_ref, bl_nb_ref,
              Wl_e_ref, bl_e_ref, Wl_from_ref, Wl_to_ref,
              x_out_ref, e_out_ref):
    f32 = jnp.float32
    dis = dis_ref[0]
    node = node_ref[0]
    dem = dem_ref[0]

    colid = lax.broadcasted_iota(jnp.int32, (NP, NP), 1)
    d = dis
    M = jnp.zeros((NP, NP), f32)
    for k in range(K + 1):
        m = jnp.min(d, axis=1, keepdims=True)
        cand = jnp.where(d == m, colid, NP)
        j = jnp.min(cand, axis=1, keepdims=True)
        sel = colid == j
        d = jnp.where(sel, jnp.inf, d)
        if k > 0:
            M = M + sel.astype(f32)
    rowid = lax.broadcasted_iota(jnp.int32, (NP, NP), 0)
    eye = (rowid == colid).astype(f32)
    a = M * (1.0 - eye) - eye

    xd = jax.nn.relu(jnp.dot(node, W1_ref[...],
                             preferred_element_type=f32) + b1_ref[...])
    xc = jax.nn.relu(jnp.dot(node, W23_ref[...], preferred_element_type=f32)
                     + dem * w3_ref[...] + b23_ref[...])
    row0 = lax.broadcasted_iota(jnp.int32, (NP, 1), 0) == 0
    x = jnp.where(row0, xd, xc)
    x = jnp.dot(x, Wn_ref[...], preferred_element_type=f32)

    a_part = (a[:, :, None] * w5_ref[...][None]).reshape(NP * NP, H)
    e0 = jax.nn.relu(disf_ref[0] * w4_ref[...] + a_part + b45_ref[...])
    e = jnp.dot(e0, We_ref[...], preferred_element_type=f32)

    inv_k = 1.0 / K
    for l in range(L):
        mx = jnp.dot(M, x, preferred_element_type=f32) * inv_k
        agg = jnp.dot(mx, Wl_nb_ref[l], preferred_element_type=f32) + bl_nb_ref[l]
        x = x + jax.nn.relu(
            jnp.dot(x, Wl_self_ref[l], preferred_element_type=f32)
            + bl_self_ref[l] + agg)
        xf = jnp.dot(x, Wl_from_ref[l], preferred_element_type=f32) + bl_e_ref[l]
        xt = jnp.dot(x, Wl_to_ref[l], preferred_element_type=f32)
        et = jnp.dot(e, Wl_e_ref[l], preferred_element_type=f32)
        e3 = et.reshape(NP, NP, H) + xf[:, None, :] + xt[None, :, :]
        e = e + jax.nn.relu(e3).reshape(NP * NP, H)

    x_out_ref[0] = x[:N]
    e_out_ref[0] = e.reshape(NP, NP, H)[:N, :N]


def kernel(node, demand, dis, W1, b1, W2, b2, W3, b3, W4, b4, W5, b5, Wn, We,
           Wl_self, bl_self, Wl_nb, bl_nb, Wl_e, bl_e, Wl_from, Wl_to):
    f32 = jnp.float32
    Hh = H // 2
    W23 = jnp.zeros((DIN, H), f32).at[:, :Hh].set(W2)
    w3 = jnp.zeros((1, H), f32).at[0, Hh:].set(W3[0])
    b23 = jnp.concatenate([b2, b3])[None]
    w4 = jnp.concatenate([W4[0], jnp.zeros((Hh,), f32)])[None]
    w5 = jnp.concatenate([jnp.zeros((Hh,), f32), W5[0]])[None]
    b45 = jnp.concatenate([b4, b5])[None]

    P = NP - N
    dis_p = jnp.pad(dis, ((0, 0), (0, P), (0, P)), constant_values=2.0)
    dis_flat = dis_p.reshape(B, NP * NP, 1)
    node_p = jnp.pad(node, ((0, 0), (0, P), (0, 0)))
    dem_p = jnp.pad(demand, ((0, 0), (0, P)))[..., None]

    rep = lambda shape: pl.BlockSpec(shape, lambda b: (0,) * len(shape))

    grid_spec = pl.GridSpec(
        grid=(B,),
        in_specs=[
            pl.BlockSpec((1, NP, DIN), lambda b: (b, 0, 0)),
            pl.BlockSpec((1, NP, 1), lambda b: (b, 0, 0)),
            pl.BlockSpec((1, NP, NP), lambda b: (b, 0, 0)),
            pl.BlockSpec((1, NP * NP, 1), lambda b: (b, 0, 0)),
            rep((DIN, H)), rep((1, H)), rep((DIN, H)), rep((1, H)),
            rep((1, H)), rep((1, H)), rep((1, H)), rep((1, H)),
            rep((H, H)), rep((H, H)),
            rep((L, H, H)), rep((L, H)), rep((L, H, H)), rep((L, H)),
            rep((L, H, H)), rep((L, H)), rep((L, H, H)), rep((L, H, H)),
        ],
        out_specs=[
            pl.BlockSpec((1, N, H), lambda b: (b, 0, 0)),
            pl.BlockSpec((1, N, N, H), lambda b: (b, 0, 0, 0)),
        ],
    )

    x_out, e_out = pl.pallas_call(
        _gcn_body,
        grid_spec=grid_spec,
        out_shape=[
            jax.ShapeDtypeStruct((B, N, H), f32),
            jax.ShapeDtypeStruct((B, N, N, H), f32),
        ],
        compiler_params=pltpu.CompilerParams(
            dimension_semantics=("arbitrary",),
        ),
    )(node_p, dem_p, dis_p, dis_flat, W1, b1[None], W23, w3, b23, w4, w5, b45,
      Wn, We, Wl_self, bl_self, Wl_nb, bl_nb, Wl_e, bl_e, Wl_from, Wl_to)
    return (x_out, e_out)


# speedup vs baseline: 1.1407x; 1.0643x over previous
import jax
import jax.numpy as jnp
from jax import lax
from jax.experimental import pallas as pl
from jax.experimental.pallas import tpu as pltpu

B, N, DIN, H, L, K = 16, 100, 2, 128, 3, 10
NP = 104


IPP = 2


def _gcn_body(node_ref, dem_ref, dis_ref, disf_ref, W1_ref, b1_ref, W23_ref,
              w3_ref, b23_ref, w4_ref, w5_ref, b45_ref, Wn_ref, We_ref,
              Wl_self_ref, bl_self_ref, Wl_nb_ref, bl_nb_ref,
              Wl_e_ref, bl_e_ref, Wl_from_ref, Wl_to_ref,
              x_out_ref, e_out_ref):
    f32 = jnp.float32
    for s in range(IPP):
        dis = dis_ref[s]
        node = node_ref[s]
        dem = dem_ref[s]

        colid = lax.broadcasted_iota(jnp.int32, (NP, NP), 1)
        d = dis
        M = jnp.zeros((NP, NP), f32)
        for k in range(K + 1):
            m = jnp.min(d, axis=1, keepdims=True)
            cand = jnp.where(d == m, colid, NP)
            j = jnp.min(cand, axis=1, keepdims=True)
            sel = colid == j
            d = jnp.where(sel, jnp.inf, d)
            if k > 0:
                M = M + sel.astype(f32)
        rowid = lax.broadcasted_iota(jnp.int32, (NP, NP), 0)
        eye = (rowid == colid).astype(f32)
        a = M * (1.0 - eye) - eye

        xd = jax.nn.relu(jnp.dot(node, W1_ref[...],
                                 preferred_element_type=f32) + b1_ref[...])
        xc = jax.nn.relu(jnp.dot(node, W23_ref[...], preferred_element_type=f32)
                         + dem * w3_ref[...] + b23_ref[...])
        row0 = lax.broadcasted_iota(jnp.int32, (NP, 1), 0) == 0
        x = jnp.where(row0, xd, xc)
        x = jnp.dot(x, Wn_ref[...], preferred_element_type=f32)

        a_part = (a[:, :, None] * w5_ref[...][None]).reshape(NP * NP, H)
        e0 = jax.nn.relu(disf_ref[s] * w4_ref[...] + a_part + b45_ref[...])
        e = jnp.dot(e0, We_ref[...], preferred_element_type=f32)

        inv_k = 1.0 / K
        for l in range(L):
            mx = jnp.dot(M, x, preferred_element_type=f32) * inv_k
            agg = jnp.dot(mx, Wl_nb_ref[l], preferred_element_type=f32) + bl_nb_ref[l]
            x = x + jax.nn.relu(
                jnp.dot(x, Wl_self_ref[l], preferred_element_type=f32)
                + bl_self_ref[l] + agg)
            xf = jnp.dot(x, Wl_from_ref[l], preferred_element_type=f32) + bl_e_ref[l]
            xt = jnp.dot(x, Wl_to_ref[l], preferred_element_type=f32)
            et = jnp.dot(e, Wl_e_ref[l], preferred_element_type=f32)
            e3 = et.reshape(NP, NP, H) + xf[:, None, :] + xt[None, :, :]
            e = e + jax.nn.relu(e3).reshape(NP * NP, H)

        x_out_ref[s] = x[:N]
        e_out_ref[s] = e.reshape(NP, NP, H)[:N, :N]


def kernel(node, demand, dis, W1, b1, W2, b2, W3, b3, W4, b4, W5, b5, Wn, We,
           Wl_self, bl_self, Wl_nb, bl_nb, Wl_e, bl_e, Wl_from, Wl_to):
    f32 = jnp.float32
    Hh = H // 2
    W23 = jnp.zeros((DIN, H), f32).at[:, :Hh].set(W2)
    w3 = jnp.zeros((1, H), f32).at[0, Hh:].set(W3[0])
    b23 = jnp.concatenate([b2, b3])[None]
    w4 = jnp.concatenate([W4[0], jnp.zeros((Hh,), f32)])[None]
    w5 = jnp.concatenate([jnp.zeros((Hh,), f32), W5[0]])[None]
    b45 = jnp.concatenate([b4, b5])[None]

    P = NP - N
    dis_p = jnp.pad(dis, ((0, 0), (0, P), (0, P)), constant_values=2.0)
    dis_flat = dis_p.reshape(B, NP * NP, 1)
    node_p = jnp.pad(node, ((0, 0), (0, P), (0, 0)))
    dem_p = jnp.pad(demand, ((0, 0), (0, P)))[..., None]

    rep = lambda shape: pl.BlockSpec(shape, lambda b: (0,) * len(shape))

    grid_spec = pl.GridSpec(
        grid=(B // IPP,),
        in_specs=[
            pl.BlockSpec((IPP, NP, DIN), lambda b: (b, 0, 0)),
            pl.BlockSpec((IPP, NP, 1), lambda b: (b, 0, 0)),
            pl.BlockSpec((IPP, NP, NP), lambda b: (b, 0, 0)),
            pl.BlockSpec((IPP, NP * NP, 1), lambda b: (b, 0, 0)),
            rep((DIN, H)), rep((1, H)), rep((DIN, H)), rep((1, H)),
            rep((1, H)), rep((1, H)), rep((1, H)), rep((1, H)),
            rep((H, H)), rep((H, H)),
            rep((L, H, H)), rep((L, H)), rep((L, H, H)), rep((L, H)),
            rep((L, H, H)), rep((L, H)), rep((L, H, H)), rep((L, H, H)),
        ],
        out_specs=[
            pl.BlockSpec((IPP, N, H), lambda b: (b, 0, 0)),
            pl.BlockSpec((IPP, N, N, H), lambda b: (b, 0, 0, 0)),
        ],
    )

    x_out, e_out = pl.pallas_call(
        _gcn_body,
        grid_spec=grid_spec,
        out_shape=[
            jax.ShapeDtypeStruct((B, N, H), f32),
            jax.ShapeDtypeStruct((B, N, N, H), f32),
        ],
        compiler_params=pltpu.CompilerParams(
            dimension_semantics=("arbitrary",),
        ),
    )(node_p, dem_p, dis_p, dis_flat, W1, b1[None], W23, w3, b23, w4, w5, b45,
      Wn, We, Wl_self, bl_self, Wl_nb, bl_nb, Wl_e, bl_e, Wl_from, Wl_to)
    return (x_out, e_out)
